# Initial kernel scaffold; baseline (speedup 1.0000x reference)
import jax
import jax.numpy as jnp
from jax.experimental import pallas as pl


def kernel(x_solvent, edge_index_solvent, edge_attr_solvent, x_solvent_batch, x_solute, edge_index_solute, edge_attr_solute, x_solute_batch, global_feat, num_graphs, sv_edge_W, sv_edge_b, sv_W1, sv_b1, sv_W2, sv_b2, sv_gamma, sv_beta, su_edge_W, su_edge_b, su_W1, su_b1, su_W2, su_b2, su_gamma, su_beta, fc_W, fc_b, out_W, out_b):
    raise NotImplementedError("write your pallas kernel here")


# speedup vs baseline: 1.1993x; 1.1993x over previous
import functools
import math

import jax
import jax.numpy as jnp
from jax import lax
from jax.experimental import pallas as pl
from jax.experimental.pallas import tpu as pltpu
from jax.experimental.pallas import tpu_sc as plsc

N = 10000
E = 320000
D = 128
DE = 16
H = 128
G = 256
L = 3

NC = 2
NS = 16
NW = NC * NS
EC = 80
CHUNKS = E // EC
CPW = CHUNKS // NW
NPT = 624
TAIL = N - NS * NPT
TAIL0 = NS * NPT
PC = 104
PCPT = NPT // PC

_BN_SCALE = 1.0 / math.sqrt(1.0 + 1e-5)

_mesh = plsc.VectorSubcoreMesh(core_axis_name="c", subcore_axis_name="s")


@functools.partial(
    pl.kernel,
    out_type=jax.ShapeDtypeStruct((NC, N, H), jnp.float32),
    mesh=_mesh,
    scratch_types=[
        pltpu.VMEM((EC,), jnp.int32),
        pltpu.VMEM((EC,), jnp.int32),
        pltpu.VMEM((EC, H), jnp.float32),
        pltpu.VMEM((EC, H), jnp.float32),
        pltpu.VMEM((PC, H), jnp.float32),
        pltpu.VMEM_SHARED((N, H), jnp.float32),
        pltpu.SemaphoreType.DMA,
    ],
)
def _sc_message(x_hbm, ep_hbm, src_hbm, dst_hbm, out_hbm,
                src_v, dst_v, xbuf, ebuf, zbuf, agg, gsem):
    c = lax.axis_index("c")
    s = lax.axis_index("s")
    wid = c * NS + s

    def _zero(i, _):
        zbuf[i // 8, pl.ds((i % 8) * 16, 16)] = jnp.zeros((16,), jnp.float32)
        return 0
    lax.fori_loop(0, PC * 8, _zero, 0)
    for j in range(PCPT):
        pltpu.sync_copy(zbuf, agg.at[pl.ds(s * NPT + j * PC, PC), :])

    @pl.when(s == NS - 1)
    def _():
        pltpu.sync_copy(zbuf.at[pl.ds(0, TAIL), :],
                        agg.at[pl.ds(TAIL0, TAIL), :])

    plsc.subcore_barrier()

    def _chunk(j, _):
        ch = wid * CPW + j
        pltpu.sync_copy(src_hbm.at[pl.ds(ch * EC, EC)], src_v)
        pltpu.sync_copy(dst_hbm.at[pl.ds(ch * EC, EC)], dst_v)
        gcp = pltpu.async_copy(x_hbm.at[src_v], xbuf, gsem)
        pltpu.sync_copy(ep_hbm.at[pl.ds(ch * EC, EC), :], ebuf)
        gcp.wait()

        def _compute(i, _):
            r = i // 8
            k = (i % 8) * 16
            ebuf[r, pl.ds(k, 16)] = jnp.maximum(
                xbuf[r, pl.ds(k, 16)] + ebuf[r, pl.ds(k, 16)], 0.0)
            return 0
        lax.fori_loop(0, EC * 8, _compute, 0)
        pltpu.sync_copy(ebuf, agg.at[dst_v], add=True)
        return 0
    lax.fori_loop(0, CPW, _chunk, 0)

    plsc.subcore_barrier()
    pltpu.sync_copy(agg.at[pl.ds(s * NPT, NPT), :],
                    out_hbm.at[c, pl.ds(s * NPT, NPT), :])

    @pl.when(s == NS - 1)
    def _():
        pltpu.sync_copy(agg.at[pl.ds(TAIL0, TAIL), :],
                        out_hbm.at[c, pl.ds(TAIL0, TAIL), :])


@functools.partial(
    pl.kernel,
    out_type=jax.ShapeDtypeStruct((NC, G, H), jnp.float32),
    mesh=_mesh,
    scratch_types=[
        pltpu.VMEM((PC, H), jnp.float32),
        pltpu.VMEM((PC,), jnp.int32),
        pltpu.VMEM((TAIL, H), jnp.float32),
        pltpu.VMEM((TAIL,), jnp.int32),
        pltpu.VMEM((NS, H), jnp.float32),
        pltpu.VMEM_SHARED((G, H), jnp.float32),
    ],
)
def _sc_pool(hsv_hbm, hsu_hbm, bsv_hbm, bsu_hbm, out_hbm,
             rowbuf, bidx, rowbuf_t, bidx_t, zbuf, gacc):
    c = lax.axis_index("c")
    s = lax.axis_index("s")

    def _zero(i, _):
        zbuf[i // 8, pl.ds((i % 8) * 16, 16)] = jnp.zeros((16,), jnp.float32)
        return 0
    lax.fori_loop(0, NS * 8, _zero, 0)
    pltpu.sync_copy(zbuf, gacc.at[pl.ds(s * NS, NS), :])
    plsc.subcore_barrier()

    def _accumulate(h_hbm, b_hbm):
        for j in range(PCPT):
            row0 = s * NPT + j * PC
            pltpu.sync_copy(h_hbm.at[pl.ds(row0, PC), :], rowbuf)
            pltpu.sync_copy(b_hbm.at[pl.ds(row0, PC)], bidx)
            pltpu.sync_copy(rowbuf, gacc.at[bidx], add=True)

        @pl.when(s == NS - 1)
        def _():
            pltpu.sync_copy(h_hbm.at[pl.ds(TAIL0, TAIL), :], rowbuf_t)
            pltpu.sync_copy(b_hbm.at[pl.ds(TAIL0, TAIL)], bidx_t)
            pltpu.sync_copy(rowbuf_t, gacc.at[bidx_t], add=True)

    @pl.when(c == 0)
    def _():
        _accumulate(hsv_hbm, bsv_hbm)

    @pl.when(c == 1)
    def _():
        _accumulate(hsu_hbm, bsu_hbm)

    plsc.subcore_barrier()
    pltpu.sync_copy(gacc.at[pl.ds(s * NS, NS), :],
                    out_hbm.at[c, pl.ds(s * NS, NS), :])


_EB = 4000


def _edge_proj_body(ea_ref, w_ref, b_ref, o0_ref, o1_ref, o2_ref):
    a = ea_ref[...]
    outs = (o0_ref, o1_ref, o2_ref)
    for l in range(L):
        outs[l][...] = (
            jnp.dot(a, w_ref[l], preferred_element_type=jnp.float32)
            + b_ref[l][None, :])


def _edge_proj(edge_attr, eW, eb):
    grid = (E // _EB,)
    return pl.pallas_call(
        _edge_proj_body,
        grid=grid,
        in_specs=[
            pl.BlockSpec((_EB, DE), lambda i: (i, 0)),
            pl.BlockSpec((L, DE, H), lambda i: (0, 0, 0)),
            pl.BlockSpec((L, H), lambda i: (0, 0)),
        ],
        out_specs=[pl.BlockSpec((_EB, H), lambda i: (i, 0))] * L,
        out_shape=[jax.ShapeDtypeStruct((E, H), jnp.float32)] * L,
    )(edge_attr, eW, eb)


_BX = 1000


def _mlp_body(x_ref, agg_ref, w1_ref, b1_ref, w2_ref, b2_ref, g_ref, be_ref,
              o_ref):
    h = x_ref[...] + agg_ref[0] + agg_ref[1]
    t = jnp.maximum(
        jnp.dot(h, w1_ref[...], preferred_element_type=jnp.float32)
        + b1_ref[...], 0.0)
    y = (jnp.dot(t, w2_ref[...], preferred_element_type=jnp.float32)
         + b2_ref[...])
    z = y * (g_ref[...] * _BN_SCALE) + be_ref[...]
    o_ref[...] = jnp.maximum(z, 0.0)


def _node_mlp(x, agg2, W1, b1, W2, b2, gamma, beta):
    grid = (N // _BX,)
    full = lambda shape: pl.BlockSpec(shape, lambda i: tuple(0 for _ in shape))
    return pl.pallas_call(
        _mlp_body,
        grid=grid,
        in_specs=[
            pl.BlockSpec((_BX, H), lambda i: (i, 0)),
            pl.BlockSpec((NC, _BX, H), lambda i: (0, i, 0)),
            full((H, H)),
            full((1, H)),
            full((H, H)),
            full((1, H)),
            full((1, H)),
            full((1, H)),
        ],
        out_specs=pl.BlockSpec((_BX, H), lambda i: (i, 0)),
        out_shape=jax.ShapeDtypeStruct((N, H), jnp.float32),
    )(x, agg2, W1, b1.reshape(1, H), W2, b2.reshape(1, H),
      gamma.reshape(1, H), beta.reshape(1, H))


def _head_body(g_ref, phys_ref, fcw_ref, fcb_ref, ow_ref, ob_ref,
               pred_ref, gc_ref):
    gc = jnp.concatenate([g_ref[0], g_ref[1], phys_ref[...]], axis=1)
    gf = jnp.maximum(
        jnp.dot(gc, fcw_ref[...], preferred_element_type=jnp.float32)
        + fcb_ref[...], 0.0)
    pred_ref[...] = (
        jnp.dot(gf, ow_ref[...], preferred_element_type=jnp.float32)
        + ob_ref[...])
    gc_ref[...] = gc


def _head(g2, phys, fc_W, fc_b, out_W, out_b):
    return pl.pallas_call(
        _head_body,
        out_shape=[
            jax.ShapeDtypeStruct((G, 1), jnp.float32),
            jax.ShapeDtypeStruct((G, 2 * H + 4), jnp.float32),
        ],
    )(g2, phys, fc_W, fc_b.reshape(1, H), out_W, out_b.reshape(1, 1))


def _backbone_pallas(x, edge_index, edge_attr, eW, eb, W1, b1, W2, b2,
                     gamma, beta):
    src = edge_index[0]
    dst = edge_index[1]
    eprojs = _edge_proj(edge_attr, eW, eb)
    for l in range(L):
        agg2 = _sc_message(x, eprojs[l], src, dst)
        x = _node_mlp(x, agg2, W1[l], b1[l], W2[l], b2[l], gamma[l], beta[l])
    return x


def kernel(x_solvent, edge_index_solvent, edge_attr_solvent, x_solvent_batch,
           x_solute, edge_index_solute, edge_attr_solute, x_solute_batch,
           global_feat, num_graphs,
           sv_edge_W, sv_edge_b, sv_W1, sv_b1, sv_W2, sv_b2, sv_gamma, sv_beta,
           su_edge_W, su_edge_b, su_W1, su_b1, su_W2, su_b2, su_gamma, su_beta,
           fc_W, fc_b, out_W, out_b):
    h_sv = _backbone_pallas(x_solvent, edge_index_solvent, edge_attr_solvent,
                            sv_edge_W, sv_edge_b, sv_W1, sv_b1, sv_W2, sv_b2,
                            sv_gamma, sv_beta)
    h_su = _backbone_pallas(x_solute, edge_index_solute, edge_attr_solute,
                            su_edge_W, su_edge_b, su_W1, su_b1, su_W2, su_b2,
                            su_gamma, su_beta)
    g2 = _sc_pool(h_sv, h_su, x_solvent_batch, x_solute_batch)
    phys = global_feat.reshape(G, -1)
    pred, g_concat = _head(g2, phys, fc_W, fc_b, out_W, out_b)
    return (pred, g_concat)


# speedup vs baseline: 1.9034x; 1.5871x over previous
import functools
import math

import jax
import jax.numpy as jnp
from jax import lax
from jax.experimental import pallas as pl
from jax.experimental.pallas import tpu as pltpu
from jax.experimental.pallas import tpu_sc as plsc

N = 10000
E = 320000
D = 128
DE = 16
H = 128
G = 256
L = 3

NC = 2
NS = 16
NW = NC * NS
EC = 80
CHUNKS = E // EC
CPW = CHUNKS // NW
NPT = 624
TAIL = N - NS * NPT
TAIL0 = NS * NPT
PC = 104
PCPT = NPT // PC

_BN_SCALE = 1.0 / math.sqrt(1.0 + 1e-5)

_mesh = plsc.VectorSubcoreMesh(core_axis_name="c", subcore_axis_name="s")


@functools.partial(
    pl.kernel,
    out_type=jax.ShapeDtypeStruct((NC, N, H), jnp.float32),
    mesh=_mesh,
    scratch_types=[
        pltpu.VMEM((2, EC), jnp.int32),
        pltpu.VMEM((2, EC), jnp.int32),
        pltpu.VMEM((2, EC, H), jnp.float32),
        pltpu.VMEM((2, EC, H), jnp.float32),
        pltpu.VMEM_SHARED((N, H), jnp.float32),
        pltpu.SemaphoreType.DMA((2,)),
        pltpu.SemaphoreType.DMA((2,)),
        pltpu.SemaphoreType.DMA((2,)),
        pltpu.SemaphoreType.DMA((2,)),
        pltpu.SemaphoreType.DMA((2,)),
    ],
)
def _sc_message(x_hbm, ep_hbm, src_hbm, dst_hbm, out_hbm,
                srcv, dstv, xbuf, mbuf, agg, srcsem, dsem, gsem, epsem, ssem):
    c = lax.axis_index("c")
    s = lax.axis_index("s")
    wid = c * NS + s
    base = wid * CPW

    def _zerofill(i, _):
        mbuf[0, i // 8, pl.ds((i % 8) * 16, 16)] = jnp.zeros((16,),
                                                             jnp.float32)
        return 0
    lax.fori_loop(0, EC * 8, _zerofill, 0)
    for j in range(NPT // EC):
        pltpu.sync_copy(mbuf.at[0], agg.at[pl.ds(s * NPT + j * EC, EC), :])
    pltpu.sync_copy(mbuf.at[0, pl.ds(0, NPT % EC), :],
                    agg.at[pl.ds(s * NPT + (NPT // EC) * EC, NPT % EC), :])

    @pl.when(s == NS - 1)
    def _():
        pltpu.sync_copy(mbuf.at[0, pl.ds(0, TAIL), :],
                        agg.at[pl.ds(TAIL0, TAIL), :])

    plsc.subcore_barrier()

    def _issue_idx(j, b):
        pltpu.async_copy(src_hbm.at[pl.ds((base + j) * EC, EC)],
                         srcv.at[b], srcsem.at[b])

    def _issue_body(j, b):
        pltpu.make_async_copy(src_hbm.at[pl.ds(0, EC)], srcv.at[b],
                              srcsem.at[b]).wait()
        pltpu.async_copy(x_hbm.at[srcv.at[b]], xbuf.at[b], gsem.at[b])
        pltpu.async_copy(ep_hbm.at[pl.ds((base + j) * EC, EC), :],
                         mbuf.at[b], epsem.at[b])
        pltpu.async_copy(dst_hbm.at[pl.ds((base + j) * EC, EC)],
                         dstv.at[b], dsem.at[b])

    def _wait_body(b):
        pltpu.make_async_copy(x_hbm.at[srcv.at[b]], xbuf.at[b],
                              gsem.at[b]).wait()
        pltpu.make_async_copy(ep_hbm.at[pl.ds(0, EC), :], mbuf.at[b],
                              epsem.at[b]).wait()
        pltpu.make_async_copy(dst_hbm.at[pl.ds(0, EC)], dstv.at[b],
                              dsem.at[b]).wait()

    def _wait_scatter(b):
        pltpu.make_async_copy(mbuf.at[b], agg.at[dstv.at[b]],
                              ssem.at[b]).wait()

    def _compute(b):
        def _row(r, _):
            for k in range(H // 16):
                sl = pl.ds(k * 16, 16)
                mbuf[b, r, sl] = jnp.maximum(
                    mbuf[b, r, sl] + xbuf[b, r, sl], 0.0)
            return 0
        lax.fori_loop(0, EC, _row, 0, unroll=2)

    _issue_idx(0, 0)
    _issue_idx(1, 1)
    _issue_body(0, 0)

    def _pair(p, _):
        for b in (0, 1):
            j = 2 * p + b
            nb = 1 - b

            @pl.when(jnp.logical_and(j >= 1, j + 1 < CPW))
            def _():
                _wait_scatter(nb)

            @pl.when(j + 1 < CPW)
            def _():
                _issue_body(j + 1, nb)

            @pl.when(j < CPW)
            def _():
                _wait_body(b)

            @pl.when(j + 2 < CPW)
            def _():
                _issue_idx(j + 2, b)

            @pl.when(j < CPW)
            def _():
                _compute(b)
                pltpu.async_copy(mbuf.at[b], agg.at[dstv.at[b]], ssem.at[b],
                                 add=True)
        return 0
    lax.fori_loop(0, (CPW + 1) // 2, _pair, 0)

    _wait_scatter((CPW - 1) % 2)
    _wait_scatter(CPW % 2)

    plsc.subcore_barrier()
    pltpu.sync_copy(agg.at[pl.ds(s * NPT, NPT), :],
                    out_hbm.at[c, pl.ds(s * NPT, NPT), :])

    @pl.when(s == NS - 1)
    def _():
        pltpu.sync_copy(agg.at[pl.ds(TAIL0, TAIL), :],
                        out_hbm.at[c, pl.ds(TAIL0, TAIL), :])


@functools.partial(
    pl.kernel,
    out_type=jax.ShapeDtypeStruct((NC, G, H), jnp.float32),
    mesh=_mesh,
    scratch_types=[
        pltpu.VMEM((PC, H), jnp.float32),
        pltpu.VMEM((PC,), jnp.int32),
        pltpu.VMEM((TAIL, H), jnp.float32),
        pltpu.VMEM((TAIL,), jnp.int32),
        pltpu.VMEM((NS, H), jnp.float32),
        pltpu.VMEM_SHARED((G, H), jnp.float32),
    ],
)
def _sc_pool(hsv_hbm, hsu_hbm, bsv_hbm, bsu_hbm, out_hbm,
             rowbuf, bidx, rowbuf_t, bidx_t, zbuf, gacc):
    c = lax.axis_index("c")
    s = lax.axis_index("s")

    def _zerofill(i, _):
        zbuf[i // 8, pl.ds((i % 8) * 16, 16)] = jnp.zeros((16,), jnp.float32)
        return 0
    lax.fori_loop(0, NS * 8, _zerofill, 0)
    pltpu.sync_copy(zbuf, gacc.at[pl.ds(s * NS, NS), :])
    plsc.subcore_barrier()

    def _accumulate(h_hbm, b_hbm):
        for j in range(PCPT):
            row0 = s * NPT + j * PC
            pltpu.sync_copy(h_hbm.at[pl.ds(row0, PC), :], rowbuf)
            pltpu.sync_copy(b_hbm.at[pl.ds(row0, PC)], bidx)
            pltpu.sync_copy(rowbuf, gacc.at[bidx], add=True)

        @pl.when(s == NS - 1)
        def _():
            pltpu.sync_copy(h_hbm.at[pl.ds(TAIL0, TAIL), :], rowbuf_t)
            pltpu.sync_copy(b_hbm.at[pl.ds(TAIL0, TAIL)], bidx_t)
            pltpu.sync_copy(rowbuf_t, gacc.at[bidx_t], add=True)

    @pl.when(c == 0)
    def _():
        _accumulate(hsv_hbm, bsv_hbm)

    @pl.when(c == 1)
    def _():
        _accumulate(hsu_hbm, bsu_hbm)

    plsc.subcore_barrier()
    pltpu.sync_copy(gacc.at[pl.ds(s * NS, NS), :],
                    out_hbm.at[c, pl.ds(s * NS, NS), :])


_EB = 4000


def _edge_proj_body(ea_ref, w_ref, b_ref, o0_ref, o1_ref, o2_ref):
    a = ea_ref[...]
    outs = (o0_ref, o1_ref, o2_ref)
    for l in range(L):
        outs[l][...] = (
            jnp.dot(a, w_ref[l], preferred_element_type=jnp.float32)
            + b_ref[l][None, :])


def _edge_proj(edge_attr, eW, eb):
    grid = (E // _EB,)
    return pl.pallas_call(
        _edge_proj_body,
        grid=grid,
        in_specs=[
            pl.BlockSpec((_EB, DE), lambda i: (i, 0)),
            pl.BlockSpec((L, DE, H), lambda i: (0, 0, 0)),
            pl.BlockSpec((L, H), lambda i: (0, 0)),
        ],
        out_specs=[pl.BlockSpec((_EB, H), lambda i: (i, 0))] * L,
        out_shape=[jax.ShapeDtypeStruct((E, H), jnp.float32)] * L,
    )(edge_attr, eW, eb)


_BX = 1000


def _mlp_body(x_ref, agg_ref, w1_ref, b1_ref, w2_ref, b2_ref, g_ref, be_ref,
              o_ref):
    h = x_ref[...] + agg_ref[0] + agg_ref[1]
    t = jnp.maximum(
        jnp.dot(h, w1_ref[...], preferred_element_type=jnp.float32)
        + b1_ref[...], 0.0)
    y = (jnp.dot(t, w2_ref[...], preferred_element_type=jnp.float32)
         + b2_ref[...])
    z = y * (g_ref[...] * _BN_SCALE) + be_ref[...]
    o_ref[...] = jnp.maximum(z, 0.0)


def _node_mlp(x, agg2, W1, b1, W2, b2, gamma, beta):
    grid = (N // _BX,)
    full = lambda shape: pl.BlockSpec(shape, lambda i: tuple(0 for _ in shape))
    return pl.pallas_call(
        _mlp_body,
        grid=grid,
        in_specs=[
            pl.BlockSpec((_BX, H), lambda i: (i, 0)),
            pl.BlockSpec((NC, _BX, H), lambda i: (0, i, 0)),
            full((H, H)),
            full((1, H)),
            full((H, H)),
            full((1, H)),
            full((1, H)),
            full((1, H)),
        ],
        out_specs=pl.BlockSpec((_BX, H), lambda i: (i, 0)),
        out_shape=jax.ShapeDtypeStruct((N, H), jnp.float32),
    )(x, agg2, W1, b1.reshape(1, H), W2, b2.reshape(1, H),
      gamma.reshape(1, H), beta.reshape(1, H))


def _head_body(g_ref, phys_ref, fcw_ref, fcb_ref, ow_ref, ob_ref,
               pred_ref, gc_ref):
    gc = jnp.concatenate([g_ref[0], g_ref[1], phys_ref[...]], axis=1)
    gf = jnp.maximum(
        jnp.dot(gc, fcw_ref[...], preferred_element_type=jnp.float32)
        + fcb_ref[...], 0.0)
    pred_ref[...] = (
        jnp.dot(gf, ow_ref[...], preferred_element_type=jnp.float32)
        + ob_ref[...])
    gc_ref[...] = gc


def _head(g2, phys, fc_W, fc_b, out_W, out_b):
    return pl.pallas_call(
        _head_body,
        out_shape=[
            jax.ShapeDtypeStruct((G, 1), jnp.float32),
            jax.ShapeDtypeStruct((G, 2 * H + 4), jnp.float32),
        ],
    )(g2, phys, fc_W, fc_b.reshape(1, H), out_W, out_b.reshape(1, 1))


def _backbone_pallas(x, edge_index, edge_attr, eW, eb, W1, b1, W2, b2,
                     gamma, beta):
    src = edge_index[0]
    dst = edge_index[1]
    eprojs = _edge_proj(edge_attr, eW, eb)
    for l in range(L):
        agg2 = _sc_message(x, eprojs[l], src, dst)
        x = _node_mlp(x, agg2, W1[l], b1[l], W2[l], b2[l], gamma[l], beta[l])
    return x


def kernel(x_solvent, edge_index_solvent, edge_attr_solvent, x_solvent_batch,
           x_solute, edge_index_solute, edge_attr_solute, x_solute_batch,
           global_feat, num_graphs,
           sv_edge_W, sv_edge_b, sv_W1, sv_b1, sv_W2, sv_b2, sv_gamma, sv_beta,
           su_edge_W, su_edge_b, su_W1, su_b1, su_W2, su_b2, su_gamma, su_beta,
           fc_W, fc_b, out_W, out_b):
    h_sv = _backbone_pallas(x_solvent, edge_index_solvent, edge_attr_solvent,
                            sv_edge_W, sv_edge_b, sv_W1, sv_b1, sv_W2, sv_b2,
                            sv_gamma, sv_beta)
    h_su = _backbone_pallas(x_solute, edge_index_solute, edge_attr_solute,
                            su_edge_W, su_edge_b, su_W1, su_b1, su_W2, su_b2,
                            su_gamma, su_beta)
    g2 = _sc_pool(h_sv, h_su, x_solvent_batch, x_solute_batch)
    phys = global_feat.reshape(G, -1)
    pred, g_concat = _head(g2, phys, fc_W, fc_b, out_W, out_b)
    return (pred, g_concat)


# speedup vs baseline: 1.9036x; 1.0001x over previous
import functools
import math

import jax
import jax.numpy as jnp
from jax import lax
from jax.experimental import pallas as pl
from jax.experimental.pallas import tpu as pltpu
from jax.experimental.pallas import tpu_sc as plsc

N = 10000
E = 320000
D = 128
DE = 16
H = 128
G = 256
L = 3

NC = 2
NS = 16
NW = NC * NS
EC = 80
CHUNKS = E // EC
CPW = CHUNKS // NW
NPT = 624
TAIL = N - NS * NPT
TAIL0 = NS * NPT
PC = 104
PCPT = NPT // PC

_BN_SCALE = 1.0 / math.sqrt(1.0 + 1e-5)

_mesh = plsc.VectorSubcoreMesh(core_axis_name="c", subcore_axis_name="s")


@functools.partial(
    pl.kernel,
    out_type=jax.ShapeDtypeStruct((NC, N, H), jnp.float32),
    mesh=_mesh,
    scratch_types=[
        pltpu.VMEM((2, EC), jnp.int32),
        pltpu.VMEM((2, EC), jnp.int32),
        pltpu.VMEM((2, EC, H), jnp.float32),
        pltpu.VMEM((2, EC, H), jnp.float32),
        pltpu.VMEM_SHARED((N, H), jnp.float32),
        pltpu.SemaphoreType.DMA((2,)),
        pltpu.SemaphoreType.DMA((2,)),
        pltpu.SemaphoreType.DMA((2,)),
        pltpu.SemaphoreType.DMA((2,)),
        pltpu.SemaphoreType.DMA((2,)),
    ],
)
def _sc_message(x_hbm, ep_hbm, src_hbm, dst_hbm, out_hbm,
                srcv, dstv, xbuf, mbuf, agg, srcsem, dsem, gsem, epsem, ssem):
    c = lax.axis_index("c")
    s = lax.axis_index("s")
    wid = c * NS + s
    base = wid * CPW

    def _zerofill(i, _):
        mbuf[0, i // 8, pl.ds((i % 8) * 16, 16)] = jnp.zeros((16,),
                                                             jnp.float32)
        return 0
    lax.fori_loop(0, EC * 8, _zerofill, 0)
    for j in range(NPT // EC):
        pltpu.sync_copy(mbuf.at[0], agg.at[pl.ds(s * NPT + j * EC, EC), :])
    pltpu.sync_copy(mbuf.at[0, pl.ds(0, NPT % EC), :],
                    agg.at[pl.ds(s * NPT + (NPT // EC) * EC, NPT % EC), :])

    @pl.when(s == NS - 1)
    def _():
        pltpu.sync_copy(mbuf.at[0, pl.ds(0, TAIL), :],
                        agg.at[pl.ds(TAIL0, TAIL), :])

    plsc.subcore_barrier()

    def _issue_idx(j, b):
        pltpu.async_copy(src_hbm.at[pl.ds((base + j) * EC, EC)],
                         srcv.at[b], srcsem.at[b])

    def _issue_body(j, b):
        pltpu.make_async_copy(src_hbm.at[pl.ds(0, EC)], srcv.at[b],
                              srcsem.at[b]).wait()
        pltpu.async_copy(x_hbm.at[srcv.at[b]], xbuf.at[b], gsem.at[b])
        pltpu.async_copy(ep_hbm.at[pl.ds((base + j) * EC, EC), :],
                         mbuf.at[b], epsem.at[b])
        pltpu.async_copy(dst_hbm.at[pl.ds((base + j) * EC, EC)],
                         dstv.at[b], dsem.at[b])

    def _wait_body(b):
        pltpu.make_async_copy(x_hbm.at[srcv.at[b]], xbuf.at[b],
                              gsem.at[b]).wait()
        pltpu.make_async_copy(ep_hbm.at[pl.ds(0, EC), :], mbuf.at[b],
                              epsem.at[b]).wait()
        pltpu.make_async_copy(dst_hbm.at[pl.ds(0, EC)], dstv.at[b],
                              dsem.at[b]).wait()

    def _wait_scatter(b):
        pltpu.make_async_copy(mbuf.at[b], agg.at[dstv.at[b]],
                              ssem.at[b]).wait()

    def _compute(b):
        def _row(r, _):
            for k in range(H // 16):
                sl = pl.ds(k * 16, 16)
                mbuf[b, r, sl] = jnp.maximum(
                    mbuf[b, r, sl] + xbuf[b, r, sl], 0.0)
            return 0
        lax.fori_loop(0, EC, _row, 0, unroll=8)

    _issue_idx(0, 0)
    _issue_idx(1, 1)
    _issue_body(0, 0)

    def _pair(p, _):
        for b in (0, 1):
            j = 2 * p + b
            nb = 1 - b

            @pl.when(jnp.logical_and(j >= 1, j + 1 < CPW))
            def _():
                _wait_scatter(nb)

            @pl.when(j + 1 < CPW)
            def _():
                _issue_body(j + 1, nb)

            @pl.when(j < CPW)
            def _():
                _wait_body(b)

            @pl.when(j + 2 < CPW)
            def _():
                _issue_idx(j + 2, b)

            @pl.when(j < CPW)
            def _():
                _compute(b)
                pltpu.async_copy(mbuf.at[b], agg.at[dstv.at[b]], ssem.at[b],
                                 add=True)
        return 0
    lax.fori_loop(0, (CPW + 1) // 2, _pair, 0)

    _wait_scatter((CPW - 1) % 2)
    _wait_scatter(CPW % 2)

    plsc.subcore_barrier()
    pltpu.sync_copy(agg.at[pl.ds(s * NPT, NPT), :],
                    out_hbm.at[c, pl.ds(s * NPT, NPT), :])

    @pl.when(s == NS - 1)
    def _():
        pltpu.sync_copy(agg.at[pl.ds(TAIL0, TAIL), :],
                        out_hbm.at[c, pl.ds(TAIL0, TAIL), :])


@functools.partial(
    pl.kernel,
    out_type=jax.ShapeDtypeStruct((NC, G, H), jnp.float32),
    mesh=_mesh,
    scratch_types=[
        pltpu.VMEM((PC, H), jnp.float32),
        pltpu.VMEM((PC,), jnp.int32),
        pltpu.VMEM((TAIL, H), jnp.float32),
        pltpu.VMEM((TAIL,), jnp.int32),
        pltpu.VMEM((NS, H), jnp.float32),
        pltpu.VMEM_SHARED((G, H), jnp.float32),
    ],
)
def _sc_pool(hsv_hbm, hsu_hbm, bsv_hbm, bsu_hbm, out_hbm,
             rowbuf, bidx, rowbuf_t, bidx_t, zbuf, gacc):
    c = lax.axis_index("c")
    s = lax.axis_index("s")

    def _zerofill(i, _):
        zbuf[i // 8, pl.ds((i % 8) * 16, 16)] = jnp.zeros((16,), jnp.float32)
        return 0
    lax.fori_loop(0, NS * 8, _zerofill, 0)
    pltpu.sync_copy(zbuf, gacc.at[pl.ds(s * NS, NS), :])
    plsc.subcore_barrier()

    def _accumulate(h_hbm, b_hbm):
        for j in range(PCPT):
            row0 = s * NPT + j * PC
            pltpu.sync_copy(h_hbm.at[pl.ds(row0, PC), :], rowbuf)
            pltpu.sync_copy(b_hbm.at[pl.ds(row0, PC)], bidx)
            pltpu.sync_copy(rowbuf, gacc.at[bidx], add=True)

        @pl.when(s == NS - 1)
        def _():
            pltpu.sync_copy(h_hbm.at[pl.ds(TAIL0, TAIL), :], rowbuf_t)
            pltpu.sync_copy(b_hbm.at[pl.ds(TAIL0, TAIL)], bidx_t)
            pltpu.sync_copy(rowbuf_t, gacc.at[bidx_t], add=True)

    @pl.when(c == 0)
    def _():
        _accumulate(hsv_hbm, bsv_hbm)

    @pl.when(c == 1)
    def _():
        _accumulate(hsu_hbm, bsu_hbm)

    plsc.subcore_barrier()
    pltpu.sync_copy(gacc.at[pl.ds(s * NS, NS), :],
                    out_hbm.at[c, pl.ds(s * NS, NS), :])


_EB = 4000


def _edge_proj_body(ea_ref, w_ref, b_ref, o0_ref, o1_ref, o2_ref):
    a = ea_ref[...]
    outs = (o0_ref, o1_ref, o2_ref)
    for l in range(L):
        outs[l][...] = (
            jnp.dot(a, w_ref[l], preferred_element_type=jnp.float32)
            + b_ref[l][None, :])


def _edge_proj(edge_attr, eW, eb):
    grid = (E // _EB,)
    return pl.pallas_call(
        _edge_proj_body,
        grid=grid,
        in_specs=[
            pl.BlockSpec((_EB, DE), lambda i: (i, 0)),
            pl.BlockSpec((L, DE, H), lambda i: (0, 0, 0)),
            pl.BlockSpec((L, H), lambda i: (0, 0)),
        ],
        out_specs=[pl.BlockSpec((_EB, H), lambda i: (i, 0))] * L,
        out_shape=[jax.ShapeDtypeStruct((E, H), jnp.float32)] * L,
    )(edge_attr, eW, eb)


_BX = 1000


def _mlp_body(x_ref, agg_ref, w1_ref, b1_ref, w2_ref, b2_ref, g_ref, be_ref,
              o_ref):
    h = x_ref[...] + agg_ref[0] + agg_ref[1]
    t = jnp.maximum(
        jnp.dot(h, w1_ref[...], preferred_element_type=jnp.float32)
        + b1_ref[...], 0.0)
    y = (jnp.dot(t, w2_ref[...], preferred_element_type=jnp.float32)
         + b2_ref[...])
    z = y * (g_ref[...] * _BN_SCALE) + be_ref[...]
    o_ref[...] = jnp.maximum(z, 0.0)


def _node_mlp(x, agg2, W1, b1, W2, b2, gamma, beta):
    grid = (N // _BX,)
    full = lambda shape: pl.BlockSpec(shape, lambda i: tuple(0 for _ in shape))
    return pl.pallas_call(
        _mlp_body,
        grid=grid,
        in_specs=[
            pl.BlockSpec((_BX, H), lambda i: (i, 0)),
            pl.BlockSpec((NC, _BX, H), lambda i: (0, i, 0)),
            full((H, H)),
            full((1, H)),
            full((H, H)),
            full((1, H)),
            full((1, H)),
            full((1, H)),
        ],
        out_specs=pl.BlockSpec((_BX, H), lambda i: (i, 0)),
        out_shape=jax.ShapeDtypeStruct((N, H), jnp.float32),
    )(x, agg2, W1, b1.reshape(1, H), W2, b2.reshape(1, H),
      gamma.reshape(1, H), beta.reshape(1, H))


def _head_body(g_ref, phys_ref, fcw_ref, fcb_ref, ow_ref, ob_ref,
               pred_ref, gc_ref):
    gc = jnp.concatenate([g_ref[0], g_ref[1], phys_ref[...]], axis=1)
    gf = jnp.maximum(
        jnp.dot(gc, fcw_ref[...], preferred_element_type=jnp.float32)
        + fcb_ref[...], 0.0)
    pred_ref[...] = (
        jnp.dot(gf, ow_ref[...], preferred_element_type=jnp.float32)
        + ob_ref[...])
    gc_ref[...] = gc


def _head(g2, phys, fc_W, fc_b, out_W, out_b):
    return pl.pallas_call(
        _head_body,
        out_shape=[
            jax.ShapeDtypeStruct((G, 1), jnp.float32),
            jax.ShapeDtypeStruct((G, 2 * H + 4), jnp.float32),
        ],
    )(g2, phys, fc_W, fc_b.reshape(1, H), out_W, out_b.reshape(1, 1))


def _backbone_pallas(x, edge_index, edge_attr, eW, eb, W1, b1, W2, b2,
                     gamma, beta):
    src = edge_index[0]
    dst = edge_index[1]
    eprojs = _edge_proj(edge_attr, eW, eb)
    for l in range(L):
        agg2 = _sc_message(x, eprojs[l], src, dst)
        x = _node_mlp(x, agg2, W1[l], b1[l], W2[l], b2[l], gamma[l], beta[l])
    return x


def kernel(x_solvent, edge_index_solvent, edge_attr_solvent, x_solvent_batch,
           x_solute, edge_index_solute, edge_attr_solute, x_solute_batch,
           global_feat, num_graphs,
           sv_edge_W, sv_edge_b, sv_W1, sv_b1, sv_W2, sv_b2, sv_gamma, sv_beta,
           su_edge_W, su_edge_b, su_W1, su_b1, su_W2, su_b2, su_gamma, su_beta,
           fc_W, fc_b, out_W, out_b):
    h_sv = _backbone_pallas(x_solvent, edge_index_solvent, edge_attr_solvent,
                            sv_edge_W, sv_edge_b, sv_W1, sv_b1, sv_W2, sv_b2,
                            sv_gamma, sv_beta)
    h_su = _backbone_pallas(x_solute, edge_index_solute, edge_attr_solute,
                            su_edge_W, su_edge_b, su_W1, su_b1, su_W2, su_b2,
                            su_gamma, su_beta)
    g2 = _sc_pool(h_sv, h_su, x_solvent_batch, x_solute_batch)
    phys = global_feat.reshape(G, -1)
    pred, g_concat = _head(g2, phys, fc_W, fc_b, out_W, out_b)
    return (pred, g_concat)


# speedup vs baseline: 3.5955x; 1.8887x over previous
import functools
import math

import jax
import jax.numpy as jnp
from jax import lax
from jax.experimental import pallas as pl
from jax.experimental.pallas import tpu as pltpu
from jax.experimental.pallas import tpu_sc as plsc

N = 10000
E = 320000
D = 128
DE = 16
H = 128
G = 256
L = 3

NC = 2
NS = 16
NW = NC * NS
EC = 80
CHUNKS = E // EC
CPW = CHUNKS // NW
NPT = 624
TAIL = N - NS * NPT
TAIL0 = NS * NPT
PC = 104
PCPT = NPT // PC

_BN_SCALE = 1.0 / math.sqrt(1.0 + 1e-5)

_mesh = plsc.VectorSubcoreMesh(core_axis_name="c", subcore_axis_name="s")


@functools.partial(
    pl.kernel,
    out_type=jax.ShapeDtypeStruct((NC, N, H), jnp.float32),
    mesh=_mesh,
    scratch_types=[
        pltpu.VMEM((2, EC), jnp.int32),
        pltpu.VMEM((2, EC), jnp.int32),
        pltpu.VMEM((2, EC, H), jnp.float32),
        pltpu.VMEM((2, EC, H), jnp.float32),
        pltpu.VMEM_SHARED((N, H), jnp.float32),
        pltpu.SemaphoreType.DMA((2,)),
        pltpu.SemaphoreType.DMA((2,)),
        pltpu.SemaphoreType.DMA((2,)),
        pltpu.SemaphoreType.DMA((2,)),
        pltpu.SemaphoreType.DMA((2,)),
    ],
)
def _sc_message(x_hbm, ep_hbm, src_hbm, dst_hbm, out_hbm,
                srcv, dstv, xbuf, mbuf, agg, srcsem, dsem, gsem, epsem, ssem):
    c = lax.axis_index("c")
    s = lax.axis_index("s")
    wid = c * NS + s
    base = wid * CPW

    def _zerofill(i, _):
        mbuf[0, i // 8, pl.ds((i % 8) * 16, 16)] = jnp.zeros((16,),
                                                             jnp.float32)
        return 0
    lax.fori_loop(0, EC * 8, _zerofill, 0)
    for j in range(NPT // EC):
        pltpu.sync_copy(mbuf.at[0], agg.at[pl.ds(s * NPT + j * EC, EC), :])
    pltpu.sync_copy(mbuf.at[0, pl.ds(0, NPT % EC), :],
                    agg.at[pl.ds(s * NPT + (NPT // EC) * EC, NPT % EC), :])

    @pl.when(s == NS - 1)
    def _():
        pltpu.sync_copy(mbuf.at[0, pl.ds(0, TAIL), :],
                        agg.at[pl.ds(TAIL0, TAIL), :])

    plsc.subcore_barrier()

    def _issue_idx(j, b):
        pltpu.async_copy(src_hbm.at[pl.ds((base + j) * EC, EC)],
                         srcv.at[b], srcsem.at[b])

    def _issue_body(j, b):
        pltpu.make_async_copy(src_hbm.at[pl.ds(0, EC)], srcv.at[b],
                              srcsem.at[b]).wait()
        pltpu.async_copy(x_hbm.at[srcv.at[b]], xbuf.at[b], gsem.at[b])
        pltpu.async_copy(ep_hbm.at[pl.ds((base + j) * EC, EC), :],
                         mbuf.at[b], epsem.at[b])
        pltpu.async_copy(dst_hbm.at[pl.ds((base + j) * EC, EC)],
                         dstv.at[b], dsem.at[b])

    def _wait_body(b):
        pltpu.make_async_copy(x_hbm.at[srcv.at[b]], xbuf.at[b],
                              gsem.at[b]).wait()
        pltpu.make_async_copy(ep_hbm.at[pl.ds(0, EC), :], mbuf.at[b],
                              epsem.at[b]).wait()
        pltpu.make_async_copy(dst_hbm.at[pl.ds(0, EC)], dstv.at[b],
                              dsem.at[b]).wait()

    def _wait_scatter(b):
        pltpu.make_async_copy(mbuf.at[b], agg.at[dstv.at[b]],
                              ssem.at[b]).wait()

    def _compute(b):
        @plsc.parallel_loop(0, EC, 1, unroll=4)
        def _row(r):
            for k in range(H // 16):
                sl = pl.ds(k * 16, 16)
                mbuf[b, r, sl] = jnp.maximum(
                    mbuf[b, r, sl] + xbuf[b, r, sl], 0.0)

    _issue_idx(0, 0)
    _issue_idx(1, 1)
    _issue_body(0, 0)

    def _pair(p, _):
        for b in (0, 1):
            j = 2 * p + b
            nb = 1 - b

            @pl.when(jnp.logical_and(j >= 1, j + 1 < CPW))
            def _():
                _wait_scatter(nb)

            @pl.when(j + 1 < CPW)
            def _():
                _issue_body(j + 1, nb)

            @pl.when(j < CPW)
            def _():
                _wait_body(b)

            @pl.when(j + 2 < CPW)
            def _():
                _issue_idx(j + 2, b)

            @pl.when(j < CPW)
            def _():
                _compute(b)
                pltpu.async_copy(mbuf.at[b], agg.at[dstv.at[b]], ssem.at[b],
                                 add=True)
        return 0
    lax.fori_loop(0, (CPW + 1) // 2, _pair, 0)

    _wait_scatter((CPW - 1) % 2)
    _wait_scatter(CPW % 2)

    plsc.subcore_barrier()
    pltpu.sync_copy(agg.at[pl.ds(s * NPT, NPT), :],
                    out_hbm.at[c, pl.ds(s * NPT, NPT), :])

    @pl.when(s == NS - 1)
    def _():
        pltpu.sync_copy(agg.at[pl.ds(TAIL0, TAIL), :],
                        out_hbm.at[c, pl.ds(TAIL0, TAIL), :])


@functools.partial(
    pl.kernel,
    out_type=jax.ShapeDtypeStruct((NC, G, H), jnp.float32),
    mesh=_mesh,
    scratch_types=[
        pltpu.VMEM((PC, H), jnp.float32),
        pltpu.VMEM((PC,), jnp.int32),
        pltpu.VMEM((TAIL, H), jnp.float32),
        pltpu.VMEM((TAIL,), jnp.int32),
        pltpu.VMEM((NS, H), jnp.float32),
        pltpu.VMEM_SHARED((G, H), jnp.float32),
    ],
)
def _sc_pool(hsv_hbm, hsu_hbm, bsv_hbm, bsu_hbm, out_hbm,
             rowbuf, bidx, rowbuf_t, bidx_t, zbuf, gacc):
    c = lax.axis_index("c")
    s = lax.axis_index("s")

    def _zerofill(i, _):
        zbuf[i // 8, pl.ds((i % 8) * 16, 16)] = jnp.zeros((16,), jnp.float32)
        return 0
    lax.fori_loop(0, NS * 8, _zerofill, 0)
    pltpu.sync_copy(zbuf, gacc.at[pl.ds(s * NS, NS), :])
    plsc.subcore_barrier()

    def _accumulate(h_hbm, b_hbm):
        for j in range(PCPT):
            row0 = s * NPT + j * PC
            pltpu.sync_copy(h_hbm.at[pl.ds(row0, PC), :], rowbuf)
            pltpu.sync_copy(b_hbm.at[pl.ds(row0, PC)], bidx)
            pltpu.sync_copy(rowbuf, gacc.at[bidx], add=True)

        @pl.when(s == NS - 1)
        def _():
            pltpu.sync_copy(h_hbm.at[pl.ds(TAIL0, TAIL), :], rowbuf_t)
            pltpu.sync_copy(b_hbm.at[pl.ds(TAIL0, TAIL)], bidx_t)
            pltpu.sync_copy(rowbuf_t, gacc.at[bidx_t], add=True)

    @pl.when(c == 0)
    def _():
        _accumulate(hsv_hbm, bsv_hbm)

    @pl.when(c == 1)
    def _():
        _accumulate(hsu_hbm, bsu_hbm)

    plsc.subcore_barrier()
    pltpu.sync_copy(gacc.at[pl.ds(s * NS, NS), :],
                    out_hbm.at[c, pl.ds(s * NS, NS), :])


_EB = 4000


def _edge_proj_body(ea_ref, w_ref, b_ref, o0_ref, o1_ref, o2_ref):
    a = ea_ref[...]
    outs = (o0_ref, o1_ref, o2_ref)
    for l in range(L):
        outs[l][...] = (
            jnp.dot(a, w_ref[l], preferred_element_type=jnp.float32)
            + b_ref[l][None, :])


def _edge_proj(edge_attr, eW, eb):
    grid = (E // _EB,)
    return pl.pallas_call(
        _edge_proj_body,
        grid=grid,
        in_specs=[
            pl.BlockSpec((_EB, DE), lambda i: (i, 0)),
            pl.BlockSpec((L, DE, H), lambda i: (0, 0, 0)),
            pl.BlockSpec((L, H), lambda i: (0, 0)),
        ],
        out_specs=[pl.BlockSpec((_EB, H), lambda i: (i, 0))] * L,
        out_shape=[jax.ShapeDtypeStruct((E, H), jnp.float32)] * L,
    )(edge_attr, eW, eb)


_BX = 1000


def _mlp_body(x_ref, agg_ref, w1_ref, b1_ref, w2_ref, b2_ref, g_ref, be_ref,
              o_ref):
    h = x_ref[...] + agg_ref[0] + agg_ref[1]
    t = jnp.maximum(
        jnp.dot(h, w1_ref[...], preferred_element_type=jnp.float32)
        + b1_ref[...], 0.0)
    y = (jnp.dot(t, w2_ref[...], preferred_element_type=jnp.float32)
         + b2_ref[...])
    z = y * (g_ref[...] * _BN_SCALE) + be_ref[...]
    o_ref[...] = jnp.maximum(z, 0.0)


def _node_mlp(x, agg2, W1, b1, W2, b2, gamma, beta):
    grid = (N // _BX,)
    full = lambda shape: pl.BlockSpec(shape, lambda i: tuple(0 for _ in shape))
    return pl.pallas_call(
        _mlp_body,
        grid=grid,
        in_specs=[
            pl.BlockSpec((_BX, H), lambda i: (i, 0)),
            pl.BlockSpec((NC, _BX, H), lambda i: (0, i, 0)),
            full((H, H)),
            full((1, H)),
            full((H, H)),
            full((1, H)),
            full((1, H)),
            full((1, H)),
        ],
        out_specs=pl.BlockSpec((_BX, H), lambda i: (i, 0)),
        out_shape=jax.ShapeDtypeStruct((N, H), jnp.float32),
    )(x, agg2, W1, b1.reshape(1, H), W2, b2.reshape(1, H),
      gamma.reshape(1, H), beta.reshape(1, H))


def _head_body(g_ref, phys_ref, fcw_ref, fcb_ref, ow_ref, ob_ref,
               pred_ref, gc_ref):
    gc = jnp.concatenate([g_ref[0], g_ref[1], phys_ref[...]], axis=1)
    gf = jnp.maximum(
        jnp.dot(gc, fcw_ref[...], preferred_element_type=jnp.float32)
        + fcb_ref[...], 0.0)
    pred_ref[...] = (
        jnp.dot(gf, ow_ref[...], preferred_element_type=jnp.float32)
        + ob_ref[...])
    gc_ref[...] = gc


def _head(g2, phys, fc_W, fc_b, out_W, out_b):
    return pl.pallas_call(
        _head_body,
        out_shape=[
            jax.ShapeDtypeStruct((G, 1), jnp.float32),
            jax.ShapeDtypeStruct((G, 2 * H + 4), jnp.float32),
        ],
    )(g2, phys, fc_W, fc_b.reshape(1, H), out_W, out_b.reshape(1, 1))


def _backbone_pallas(x, edge_index, edge_attr, eW, eb, W1, b1, W2, b2,
                     gamma, beta):
    src = edge_index[0]
    dst = edge_index[1]
    eprojs = _edge_proj(edge_attr, eW, eb)
    for l in range(L):
        agg2 = _sc_message(x, eprojs[l], src, dst)
        x = _node_mlp(x, agg2, W1[l], b1[l], W2[l], b2[l], gamma[l], beta[l])
    return x


def kernel(x_solvent, edge_index_solvent, edge_attr_solvent, x_solvent_batch,
           x_solute, edge_index_solute, edge_attr_solute, x_solute_batch,
           global_feat, num_graphs,
           sv_edge_W, sv_edge_b, sv_W1, sv_b1, sv_W2, sv_b2, sv_gamma, sv_beta,
           su_edge_W, su_edge_b, su_W1, su_b1, su_W2, su_b2, su_gamma, su_beta,
           fc_W, fc_b, out_W, out_b):
    h_sv = _backbone_pallas(x_solvent, edge_index_solvent, edge_attr_solvent,
                            sv_edge_W, sv_edge_b, sv_W1, sv_b1, sv_W2, sv_b2,
                            sv_gamma, sv_beta)
    h_su = _backbone_pallas(x_solute, edge_index_solute, edge_attr_solute,
                            su_edge_W, su_edge_b, su_W1, su_b1, su_W2, su_b2,
                            su_gamma, su_beta)
    g2 = _sc_pool(h_sv, h_su, x_solvent_batch, x_solute_batch)
    phys = global_feat.reshape(G, -1)
    pred, g_concat = _head(g2, phys, fc_W, fc_b, out_W, out_b)
    return (pred, g_concat)
